# Initial kernel scaffold; baseline (speedup 1.0000x reference)
import jax
import jax.numpy as jnp
from jax.experimental import pallas as pl


def kernel(node_features, edge_index, W, b, gamma, beta):
    raise NotImplementedError("write your pallas kernel here")


# speedup vs baseline: 20.0731x; 20.0731x over previous
import functools
import math

import jax
import jax.numpy as jnp
from jax import lax
from jax.experimental import pallas as pl
from jax.experimental.pallas import tpu as pltpu
from jax.experimental.pallas import tpu_sc as plsc

EPS = 1e-3
NC, NS = 2, 16
NW = NC * NS
CH = 80


def _deg_pallas(src, zeros_hbm, n, npad, e):
    ept = e // NW
    nchunk = ept // CH
    rpt = npad // NS
    mesh = plsc.VectorSubcoreMesh(core_axis_name="c", subcore_axis_name="s")

    @functools.partial(
        pl.kernel,
        out_type=jax.ShapeDtypeStruct((NC * npad, 128), jnp.float32),
        mesh=mesh,
        scratch_types=[
            pltpu.VMEM((CH,), jnp.int32),
            pltpu.VMEM((CH, 128), jnp.float32),
            pltpu.VMEM_SHARED((npad, 128), jnp.float32),
        ],
    )
    def deg_kernel(src_hbm, z_hbm, out_hbm, idx_v, ones_v, acc):
        c = lax.axis_index("c")
        s = lax.axis_index("s")
        wid = c * NS + s

        def fill_ones(i, carry):
            for j in range(8):
                ones_v[i, pl.ds(j * 16, 16)] = jnp.ones((16,), jnp.float32)
            return carry

        lax.fori_loop(0, CH, fill_ones, 0)
        pltpu.sync_copy(z_hbm.at[pl.ds(s * rpt, rpt)], acc.at[pl.ds(s * rpt, rpt)])
        plsc.subcore_barrier()

        def body(i, carry):
            base = pl.multiple_of(wid * ept + i * CH, 8)
            pltpu.sync_copy(src_hbm.at[pl.ds(base, CH)], idx_v)
            pltpu.sync_copy(ones_v, acc.at[idx_v], add=True)
            return carry

        lax.fori_loop(0, nchunk, body, 0)
        plsc.subcore_barrier()
        pltpu.sync_copy(acc.at[pl.ds(s * rpt, rpt)],
                        out_hbm.at[pl.ds(c * npad + s * rpt, rpt)])

    return deg_kernel(src, zeros_hbm).reshape(NC, npad, 128)


def _agg_pallas(y, dst, src, zeros_hbm, n, npad, e, h):
    ept = e // NW
    nchunk = ept // CH
    rpt = npad // NS
    mesh = plsc.VectorSubcoreMesh(core_axis_name="c", subcore_axis_name="s")

    @functools.partial(
        pl.kernel,
        out_type=jax.ShapeDtypeStruct((NC * npad, h), jnp.float32),
        mesh=mesh,
        scratch_types=[
            pltpu.VMEM((CH,), jnp.int32),
            pltpu.VMEM((CH,), jnp.int32),
            pltpu.VMEM((CH, h), jnp.float32),
            pltpu.VMEM_SHARED((npad, h), jnp.float32),
            pltpu.SemaphoreType.DMA,
        ],
    )
    def agg_kernel(y_hbm, dst_hbm, src_hbm, z_hbm, out_hbm,
                   dsti_v, srci_v, rows_v, acc, gsem):
        c = lax.axis_index("c")
        s = lax.axis_index("s")
        wid = c * NS + s

        pltpu.sync_copy(z_hbm.at[pl.ds(s * rpt, rpt)], acc.at[pl.ds(s * rpt, rpt)])
        plsc.subcore_barrier()

        def body(i, carry):
            base = pl.multiple_of(wid * ept + i * CH, 8)
            pltpu.sync_copy(dst_hbm.at[pl.ds(base, CH)], dsti_v)
            pltpu.sync_copy(src_hbm.at[pl.ds(base, CH)], srci_v)
            pltpu.async_copy(y_hbm.at[dsti_v], rows_v, gsem).wait()
            pltpu.sync_copy(rows_v, acc.at[srci_v], add=True)
            return carry

        lax.fori_loop(0, nchunk, body, 0)
        plsc.subcore_barrier()
        pltpu.sync_copy(acc.at[pl.ds(s * rpt, rpt)],
                        out_hbm.at[pl.ds(c * npad + s * rpt, rpt)])

    return agg_kernel(y, dst, src, zeros_hbm).reshape(NC, npad, h)


def _transform_pallas(x, w, b, deg_parts, n, d, h):
    rb = 1000
    nblk = n // rb

    def body(x_ref, w_ref, b_ref, d0_ref, d1_ref, y_ref, u_ref):
        t = jnp.dot(x_ref[...], w_ref[...],
                    preferred_element_type=jnp.float32) + b_ref[...]
        deg = (d0_ref[...][0] + d1_ref[...][0])[:, 0:1]
        y_ref[...] = t * lax.rsqrt(deg)
        u_ref[...] = (0.5 * deg) * t

    return pl.pallas_call(
        body,
        grid=(nblk,),
        in_specs=[
            pl.BlockSpec((rb, d), lambda i: (i, 0)),
            pl.BlockSpec((d, h), lambda i: (0, 0)),
            pl.BlockSpec((1, h), lambda i: (0, 0)),
            pl.BlockSpec((1, rb, 128), lambda i: (0, i, 0)),
            pl.BlockSpec((1, rb, 128), lambda i: (1, i, 0)),
        ],
        out_specs=[pl.BlockSpec((rb, h), lambda i: (i, 0))] * 2,
        out_shape=[jax.ShapeDtypeStruct((n, h), jnp.float32)] * 2,
    )(x, w, b.reshape(1, h), deg_parts, deg_parts)


def _combine_pallas(msg_parts, u, deg_parts, gamma, beta, n, h):
    rb = 1000
    nblk = n // rb
    inv_bn = 1.0 / math.sqrt(1.0 + EPS)

    def body(m0_ref, m1_ref, u_ref, d0_ref, d1_ref, g_ref, b_ref, o_ref):
        deg = (d0_ref[...][0] + d1_ref[...][0])[:, 0:1]
        scale = jnp.where(deg > 0, 0.5 * lax.rsqrt(deg), 0.0)
        m = m0_ref[...][0] + m1_ref[...][0]
        o_ref[...] = (m * scale + u_ref[...]) * (g_ref[...] * inv_bn) + b_ref[...]

    return pl.pallas_call(
        body,
        grid=(nblk,),
        in_specs=[
            pl.BlockSpec((1, rb, h), lambda i: (0, i, 0)),
            pl.BlockSpec((1, rb, h), lambda i: (1, i, 0)),
            pl.BlockSpec((rb, h), lambda i: (i, 0)),
            pl.BlockSpec((1, rb, 128), lambda i: (0, i, 0)),
            pl.BlockSpec((1, rb, 128), lambda i: (1, i, 0)),
            pl.BlockSpec((1, h), lambda i: (0, 0)),
            pl.BlockSpec((1, h), lambda i: (0, 0)),
        ],
        out_specs=pl.BlockSpec((rb, h), lambda i: (i, 0)),
        out_shape=jax.ShapeDtypeStruct((n, h), jnp.float32),
    )(msg_parts, msg_parts, u, deg_parts, deg_parts,
      gamma.reshape(1, h), beta.reshape(1, h))


def kernel(node_features, edge_index, W, b, gamma, beta):
    n, d = node_features.shape
    e = edge_index.shape[0]
    h = W.shape[1]
    npad = ((n + 128 * NS - 1) // (128 * NS)) * (128 * NS)
    src = edge_index[:, 0].astype(jnp.int32)
    dst = edge_index[:, 1].astype(jnp.int32)
    zeros_hbm = jnp.zeros((npad, 128), jnp.float32)

    deg_parts = _deg_pallas(src, zeros_hbm, n, npad, e)
    y, u = _transform_pallas(node_features, W, b, deg_parts, n, d, h)
    msg_parts = _agg_pallas(y, dst, src, zeros_hbm, n, npad, e, h)
    return _combine_pallas(msg_parts, u, deg_parts, gamma, beta, n, h)


# speedup vs baseline: 31.3706x; 1.5628x over previous
import functools
import math

import jax
import jax.numpy as jnp
from jax import lax
from jax.experimental import pallas as pl
from jax.experimental.pallas import tpu as pltpu
from jax.experimental.pallas import tpu_sc as plsc

EPS = 1e-3
NC, NS = 2, 16
NW = NC * NS
CH = 80


def _deg_pallas(src, zeros_hbm, n, npad, e):
    ept = e // NW
    nchunk = ept // CH
    rpt = npad // NS
    mesh = plsc.VectorSubcoreMesh(core_axis_name="c", subcore_axis_name="s")

    assert nchunk % 2 == 1 and nchunk >= 3

    @functools.partial(
        pl.kernel,
        out_type=jax.ShapeDtypeStruct((NC * npad, 128), jnp.float32),
        mesh=mesh,
        scratch_types=[
            pltpu.VMEM((2, CH), jnp.int32),
            pltpu.VMEM((CH, 128), jnp.float32),
            pltpu.VMEM_SHARED((npad, 128), jnp.float32),
            pltpu.SemaphoreType.DMA,
            pltpu.SemaphoreType.DMA,
        ],
    )
    def deg_kernel(src_hbm, z_hbm, out_hbm, idx_v, ones_v, acc, isem0, isem1):
        c = lax.axis_index("c")
        s = lax.axis_index("s")
        wid = c * NS + s
        sems = (isem0, isem1)

        def load(i, slot):
            base = pl.multiple_of(wid * ept + i * CH, 8)
            pltpu.async_copy(src_hbm.at[pl.ds(base, CH)], idx_v.at[slot],
                             sems[slot])

        def finish(i, slot):
            base = pl.multiple_of(wid * ept + i * CH, 8)
            pltpu.make_async_copy(src_hbm.at[pl.ds(base, CH)], idx_v.at[slot],
                                  sems[slot]).wait()
            pltpu.sync_copy(ones_v, acc.at[idx_v.at[slot]], add=True)

        def fill_ones(i, carry):
            for j in range(8):
                ones_v[i, pl.ds(j * 16, 16)] = jnp.ones((16,), jnp.float32)
            return carry

        lax.fori_loop(0, CH, fill_ones, 0)
        pltpu.sync_copy(z_hbm.at[pl.ds(s * rpt, rpt)], acc.at[pl.ds(s * rpt, rpt)])
        plsc.subcore_barrier()

        load(0, 0)

        def body(g, carry):
            i = g * 2
            load(i + 1, 1)
            finish(i, 0)
            load(i + 2, 0)
            finish(i + 1, 1)
            return carry

        lax.fori_loop(0, (nchunk - 1) // 2, body, 0)
        finish(nchunk - 1, 0)
        plsc.subcore_barrier()
        pltpu.sync_copy(acc.at[pl.ds(s * rpt, rpt)],
                        out_hbm.at[pl.ds(c * npad + s * rpt, rpt)])

    return deg_kernel(src, zeros_hbm).reshape(NC, npad, 128)


def _agg_pallas(y, dst, src, zeros_hbm, n, npad, e, h):
    ept = e // NW
    nchunk = ept // CH
    rpt = npad // NS
    mesh = plsc.VectorSubcoreMesh(core_axis_name="c", subcore_axis_name="s")

    assert nchunk % 2 == 1 and nchunk >= 3

    @functools.partial(
        pl.kernel,
        out_type=jax.ShapeDtypeStruct((NC * npad, h), jnp.float32),
        mesh=mesh,
        scratch_types=[
            pltpu.VMEM((2, CH), jnp.int32),
            pltpu.VMEM((2, CH), jnp.int32),
            pltpu.VMEM((2, CH, h), jnp.float32),
            pltpu.VMEM_SHARED((npad, h), jnp.float32),
            pltpu.SemaphoreType.DMA,
            pltpu.SemaphoreType.DMA,
        ],
    )
    def agg_kernel(y_hbm, dst_hbm, src_hbm, z_hbm, out_hbm,
                   dsti_v, srci_v, rows_v, acc, gsem0, gsem1):
        c = lax.axis_index("c")
        s = lax.axis_index("s")
        wid = c * NS + s
        sems = (gsem0, gsem1)

        def load_and_gather(i, slot):
            base = pl.multiple_of(wid * ept + i * CH, 8)
            pltpu.sync_copy(dst_hbm.at[pl.ds(base, CH)], dsti_v.at[slot])
            pltpu.sync_copy(src_hbm.at[pl.ds(base, CH)], srci_v.at[slot])
            pltpu.async_copy(y_hbm.at[dsti_v.at[slot]], rows_v.at[slot],
                             sems[slot])

        def finish(slot):
            pltpu.make_async_copy(y_hbm.at[dsti_v.at[slot]], rows_v.at[slot],
                                  sems[slot]).wait()
            pltpu.sync_copy(rows_v.at[slot], acc.at[srci_v.at[slot]], add=True)

        pltpu.sync_copy(z_hbm.at[pl.ds(s * rpt, rpt)], acc.at[pl.ds(s * rpt, rpt)])
        plsc.subcore_barrier()

        load_and_gather(0, 0)

        def body(g, carry):
            i = g * 2
            load_and_gather(i + 1, 1)
            finish(0)
            load_and_gather(i + 2, 0)
            finish(1)
            return carry

        lax.fori_loop(0, (nchunk - 1) // 2, body, 0)
        finish(0)
        plsc.subcore_barrier()
        pltpu.sync_copy(acc.at[pl.ds(s * rpt, rpt)],
                        out_hbm.at[pl.ds(c * npad + s * rpt, rpt)])

    return agg_kernel(y, dst, src, zeros_hbm).reshape(NC, npad, h)


def _transform_pallas(x, w, b, deg_parts, n, d, h):
    rb = 1000
    nblk = n // rb

    def body(x_ref, w_ref, b_ref, d0_ref, d1_ref, y_ref, u_ref):
        t = jnp.dot(x_ref[...], w_ref[...],
                    preferred_element_type=jnp.float32) + b_ref[...]
        deg = (d0_ref[...][0] + d1_ref[...][0])[:, 0:1]
        y_ref[...] = t * lax.rsqrt(deg)
        u_ref[...] = (0.5 * deg) * t

    return pl.pallas_call(
        body,
        grid=(nblk,),
        in_specs=[
            pl.BlockSpec((rb, d), lambda i: (i, 0)),
            pl.BlockSpec((d, h), lambda i: (0, 0)),
            pl.BlockSpec((1, h), lambda i: (0, 0)),
            pl.BlockSpec((1, rb, 128), lambda i: (0, i, 0)),
            pl.BlockSpec((1, rb, 128), lambda i: (1, i, 0)),
        ],
        out_specs=[pl.BlockSpec((rb, h), lambda i: (i, 0))] * 2,
        out_shape=[jax.ShapeDtypeStruct((n, h), jnp.float32)] * 2,
    )(x, w, b.reshape(1, h), deg_parts, deg_parts)


def _combine_pallas(msg_parts, u, deg_parts, gamma, beta, n, h):
    rb = 1000
    nblk = n // rb
    inv_bn = 1.0 / math.sqrt(1.0 + EPS)

    def body(m0_ref, m1_ref, u_ref, d0_ref, d1_ref, g_ref, b_ref, o_ref):
        deg = (d0_ref[...][0] + d1_ref[...][0])[:, 0:1]
        scale = jnp.where(deg > 0, 0.5 * lax.rsqrt(deg), 0.0)
        m = m0_ref[...][0] + m1_ref[...][0]
        o_ref[...] = (m * scale + u_ref[...]) * (g_ref[...] * inv_bn) + b_ref[...]

    return pl.pallas_call(
        body,
        grid=(nblk,),
        in_specs=[
            pl.BlockSpec((1, rb, h), lambda i: (0, i, 0)),
            pl.BlockSpec((1, rb, h), lambda i: (1, i, 0)),
            pl.BlockSpec((rb, h), lambda i: (i, 0)),
            pl.BlockSpec((1, rb, 128), lambda i: (0, i, 0)),
            pl.BlockSpec((1, rb, 128), lambda i: (1, i, 0)),
            pl.BlockSpec((1, h), lambda i: (0, 0)),
            pl.BlockSpec((1, h), lambda i: (0, 0)),
        ],
        out_specs=pl.BlockSpec((rb, h), lambda i: (i, 0)),
        out_shape=jax.ShapeDtypeStruct((n, h), jnp.float32),
    )(msg_parts, msg_parts, u, deg_parts, deg_parts,
      gamma.reshape(1, h), beta.reshape(1, h))


def kernel(node_features, edge_index, W, b, gamma, beta):
    n, d = node_features.shape
    e = edge_index.shape[0]
    h = W.shape[1]
    npad = ((n + 128 * NS - 1) // (128 * NS)) * (128 * NS)
    src = edge_index[:, 0].astype(jnp.int32)
    dst = edge_index[:, 1].astype(jnp.int32)
    zeros_hbm = jnp.zeros((npad, 128), jnp.float32)

    deg_parts = _deg_pallas(src, zeros_hbm, n, npad, e)
    y, u = _transform_pallas(node_features, W, b, deg_parts, n, d, h)
    msg_parts = _agg_pallas(y, dst, src, zeros_hbm, n, npad, e, h)
    return _combine_pallas(msg_parts, u, deg_parts, gamma, beta, n, h)


# speedup vs baseline: 31.4950x; 1.0040x over previous
import functools
import math

import jax
import jax.numpy as jnp
from jax import lax
from jax.experimental import pallas as pl
from jax.experimental.pallas import tpu as pltpu
from jax.experimental.pallas import tpu_sc as plsc

EPS = 1e-3
NC, NS = 2, 16
NW = NC * NS
CH = 80


def _deg_pallas(src, zeros_hbm, n, npad, e):
    ept = e // NW
    nchunk = ept // CH
    rpt = npad // NS
    mesh = plsc.VectorSubcoreMesh(core_axis_name="c", subcore_axis_name="s")

    assert nchunk % 2 == 1 and nchunk >= 3

    @functools.partial(
        pl.kernel,
        out_type=jax.ShapeDtypeStruct((NC * npad, 128), jnp.float32),
        mesh=mesh,
        scratch_types=[
            pltpu.VMEM((2, CH), jnp.int32),
            pltpu.VMEM((CH, 128), jnp.float32),
            pltpu.VMEM_SHARED((npad, 128), jnp.float32),
            pltpu.SemaphoreType.DMA,
            pltpu.SemaphoreType.DMA,
        ],
    )
    def deg_kernel(src_hbm, z_hbm, out_hbm, idx_v, ones_v, acc, isem0, isem1):
        c = lax.axis_index("c")
        s = lax.axis_index("s")
        wid = c * NS + s
        sems = (isem0, isem1)

        def load(i, slot):
            base = pl.multiple_of(wid * ept + i * CH, 8)
            pltpu.async_copy(src_hbm.at[pl.ds(base, CH)], idx_v.at[slot],
                             sems[slot])

        def finish(i, slot):
            base = pl.multiple_of(wid * ept + i * CH, 8)
            pltpu.make_async_copy(src_hbm.at[pl.ds(base, CH)], idx_v.at[slot],
                                  sems[slot]).wait()
            pltpu.sync_copy(ones_v, acc.at[idx_v.at[slot]], add=True)

        def fill_ones(i, carry):
            for j in range(8):
                ones_v[i, pl.ds(j * 16, 16)] = jnp.ones((16,), jnp.float32)
            return carry

        lax.fori_loop(0, CH, fill_ones, 0)
        pltpu.sync_copy(z_hbm.at[pl.ds(s * rpt, rpt)], acc.at[pl.ds(s * rpt, rpt)])
        plsc.subcore_barrier()

        load(0, 0)

        def body(g, carry):
            i = g * 2
            load(i + 1, 1)
            finish(i, 0)
            load(i + 2, 0)
            finish(i + 1, 1)
            return carry

        lax.fori_loop(0, (nchunk - 1) // 2, body, 0)
        finish(nchunk - 1, 0)
        plsc.subcore_barrier()
        pltpu.sync_copy(acc.at[pl.ds(s * rpt, rpt)],
                        out_hbm.at[pl.ds(c * npad + s * rpt, rpt)])

    return deg_kernel(src, zeros_hbm).reshape(NC, npad, 128)


def _agg_pallas(y, dst, src, zeros_hbm, n, npad, e, h):
    ept = e // NW
    nchunk = ept // CH
    rpt = npad // NS
    mesh = plsc.VectorSubcoreMesh(core_axis_name="c", subcore_axis_name="s")

    nbuf = 4
    assert nchunk >= nbuf

    @functools.partial(
        pl.kernel,
        out_type=jax.ShapeDtypeStruct((NC * npad, h), jnp.float32),
        mesh=mesh,
        scratch_types=[
            pltpu.VMEM((nbuf, CH), jnp.int32),
            pltpu.VMEM((nbuf, CH), jnp.int32),
            pltpu.VMEM((nbuf, CH, h), jnp.float32),
            pltpu.VMEM_SHARED((npad, h), jnp.float32),
        ] + [pltpu.SemaphoreType.DMA] * nbuf,
    )
    def agg_kernel(y_hbm, dst_hbm, src_hbm, z_hbm, out_hbm,
                   dsti_v, srci_v, rows_v, acc, *sems):
        c = lax.axis_index("c")
        s = lax.axis_index("s")
        wid = c * NS + s

        def load_and_gather(i, slot):
            base = pl.multiple_of(wid * ept + i * CH, 8)
            pltpu.sync_copy(dst_hbm.at[pl.ds(base, CH)], dsti_v.at[slot])
            pltpu.sync_copy(src_hbm.at[pl.ds(base, CH)], srci_v.at[slot])
            pltpu.async_copy(y_hbm.at[dsti_v.at[slot]], rows_v.at[slot],
                             sems[slot])

        def finish(slot):
            pltpu.make_async_copy(y_hbm.at[dsti_v.at[slot]], rows_v.at[slot],
                                  sems[slot]).wait()
            pltpu.sync_copy(rows_v.at[slot], acc.at[srci_v.at[slot]], add=True)

        pltpu.sync_copy(z_hbm.at[pl.ds(s * rpt, rpt)], acc.at[pl.ds(s * rpt, rpt)])
        plsc.subcore_barrier()

        for k in range(nbuf - 1):
            load_and_gather(k, k)

        nmain = (nchunk - (nbuf - 1) - 1) // nbuf
        if nmain > 0:
            def body(g, carry):
                i = g * nbuf
                for b in range(nbuf):
                    load_and_gather(i + b + nbuf - 1, (b + nbuf - 1) % nbuf)
                    finish(b)
                return carry

            lax.fori_loop(0, nmain, body, 0)
        for i in range(nbuf * nmain, nchunk):
            if i + nbuf - 1 < nchunk:
                load_and_gather(i + nbuf - 1, (i + nbuf - 1) % nbuf)
            finish(i % nbuf)
        plsc.subcore_barrier()
        pltpu.sync_copy(acc.at[pl.ds(s * rpt, rpt)],
                        out_hbm.at[pl.ds(c * npad + s * rpt, rpt)])

    return agg_kernel(y, dst, src, zeros_hbm).reshape(NC, npad, h)


def _transform_pallas(x, w, b, deg_parts, n, d, h):
    rb = 1000
    nblk = n // rb

    def body(x_ref, w_ref, b_ref, d0_ref, d1_ref, y_ref, u_ref):
        t = jnp.dot(x_ref[...], w_ref[...],
                    preferred_element_type=jnp.float32) + b_ref[...]
        deg = (d0_ref[...][0] + d1_ref[...][0])[:, 0:1]
        y_ref[...] = t * lax.rsqrt(deg)
        u_ref[...] = (0.5 * deg) * t

    return pl.pallas_call(
        body,
        grid=(nblk,),
        in_specs=[
            pl.BlockSpec((rb, d), lambda i: (i, 0)),
            pl.BlockSpec((d, h), lambda i: (0, 0)),
            pl.BlockSpec((1, h), lambda i: (0, 0)),
            pl.BlockSpec((1, rb, 128), lambda i: (0, i, 0)),
            pl.BlockSpec((1, rb, 128), lambda i: (1, i, 0)),
        ],
        out_specs=[pl.BlockSpec((rb, h), lambda i: (i, 0))] * 2,
        out_shape=[jax.ShapeDtypeStruct((n, h), jnp.float32)] * 2,
    )(x, w, b.reshape(1, h), deg_parts, deg_parts)


def _combine_pallas(msg_parts, u, deg_parts, gamma, beta, n, h):
    rb = 1000
    nblk = n // rb
    inv_bn = 1.0 / math.sqrt(1.0 + EPS)

    def body(m0_ref, m1_ref, u_ref, d0_ref, d1_ref, g_ref, b_ref, o_ref):
        deg = (d0_ref[...][0] + d1_ref[...][0])[:, 0:1]
        scale = jnp.where(deg > 0, 0.5 * lax.rsqrt(deg), 0.0)
        m = m0_ref[...][0] + m1_ref[...][0]
        o_ref[...] = (m * scale + u_ref[...]) * (g_ref[...] * inv_bn) + b_ref[...]

    return pl.pallas_call(
        body,
        grid=(nblk,),
        in_specs=[
            pl.BlockSpec((1, rb, h), lambda i: (0, i, 0)),
            pl.BlockSpec((1, rb, h), lambda i: (1, i, 0)),
            pl.BlockSpec((rb, h), lambda i: (i, 0)),
            pl.BlockSpec((1, rb, 128), lambda i: (0, i, 0)),
            pl.BlockSpec((1, rb, 128), lambda i: (1, i, 0)),
            pl.BlockSpec((1, h), lambda i: (0, 0)),
            pl.BlockSpec((1, h), lambda i: (0, 0)),
        ],
        out_specs=pl.BlockSpec((rb, h), lambda i: (i, 0)),
        out_shape=jax.ShapeDtypeStruct((n, h), jnp.float32),
    )(msg_parts, msg_parts, u, deg_parts, deg_parts,
      gamma.reshape(1, h), beta.reshape(1, h))


def kernel(node_features, edge_index, W, b, gamma, beta):
    n, d = node_features.shape
    e = edge_index.shape[0]
    h = W.shape[1]
    npad = ((n + 128 * NS - 1) // (128 * NS)) * (128 * NS)
    src = edge_index[:, 0].astype(jnp.int32)
    dst = edge_index[:, 1].astype(jnp.int32)
    zeros_hbm = jnp.zeros((npad, 128), jnp.float32)

    deg_parts = _deg_pallas(src, zeros_hbm, n, npad, e)
    y, u = _transform_pallas(node_features, W, b, deg_parts, n, d, h)
    msg_parts = _agg_pallas(y, dst, src, zeros_hbm, n, npad, e, h)
    return _combine_pallas(msg_parts, u, deg_parts, gamma, beta, n, h)
